# Initial kernel scaffold; baseline (speedup 1.0000x reference)
import jax
import jax.numpy as jnp
from jax.experimental import pallas as pl


def kernel(x, scale0_params, scale1_params, scale2_params, fusion_params):
    raise NotImplementedError("write your pallas kernel here")


# speedup vs baseline: 1.9917x; 1.9917x over previous
import jax
import jax.numpy as jnp
from jax.experimental import pallas as pl
from jax.experimental.pallas import tpu as pltpu

_TILE_N = 512


def _fused_mlp_kernel(x_ref, w1_ref, b1_ref, w2_ref, b2_ref, w3_ref, b3_ref,
                      o_ref):
    x = x_ref[0]
    dot = lambda w, h: jax.lax.dot_general(
        w, h, (((1,), (0,)), ((), ())), preferred_element_type=jnp.float32)
    h = jnp.maximum(dot(w1_ref[...], x) + b1_ref[...], 0.0)
    h = jnp.maximum(dot(w2_ref[...], h) + b2_ref[...], 0.0)
    o_ref[0] = jnp.maximum(dot(w3_ref[...], h) + b3_ref[...], 0.0)


def kernel(x, scale0_params, scale1_params, scale2_params, fusion_params):
    del scale0_params, scale1_params, scale2_params
    (w1, b1), (w2, b2), (w3, b3) = fusion_params
    B, C_in, N = x.shape
    C_out = w3.shape[0]
    tile_n = min(_TILE_N, N)
    grid = (B, N // tile_n)

    full = lambda shape: pl.BlockSpec(shape, lambda b, n: (0,) * len(shape))
    return pl.pallas_call(
        _fused_mlp_kernel,
        grid=grid,
        in_specs=[
            pl.BlockSpec((1, C_in, tile_n), lambda b, n: (b, 0, n)),
            full(w1.shape), full((w1.shape[0], 1)),
            full(w2.shape), full((w2.shape[0], 1)),
            full(w3.shape), full((w3.shape[0], 1)),
        ],
        out_specs=pl.BlockSpec((1, C_out, tile_n), lambda b, n: (b, 0, n)),
        out_shape=jax.ShapeDtypeStruct((B, C_out, N), jnp.float32),
        compiler_params=pltpu.CompilerParams(
            dimension_semantics=("parallel", "parallel")),
    )(x, w1, b1[:, None], w2, b2[:, None], w3, b3[:, None])


# speedup vs baseline: 2.5223x; 1.2664x over previous
import jax
import jax.numpy as jnp
from jax.experimental import pallas as pl
from jax.experimental.pallas import tpu as pltpu

_TILE_N = 1024


def _fused_mlp_kernel(x_ref, w1_ref, b1_ref, w2_ref, b2_ref, w3_ref, b3_ref,
                      o_ref):
    x = x_ref[0]
    dot = lambda w, h: jax.lax.dot_general(
        w, h, (((1,), (0,)), ((), ())), preferred_element_type=jnp.float32)
    h = jnp.maximum(dot(w1_ref[...], x) + b1_ref[...], 0.0)
    h = jnp.maximum(dot(w2_ref[...], h) + b2_ref[...], 0.0)
    o_ref[0] = jnp.maximum(dot(w3_ref[...], h) + b3_ref[...], 0.0)


def kernel(x, scale0_params, scale1_params, scale2_params, fusion_params):
    del scale0_params, scale1_params, scale2_params
    (w1, b1), (w2, b2), (w3, b3) = fusion_params
    B, C_in, N = x.shape
    C_out = w3.shape[0]
    tile_n = min(_TILE_N, N)
    grid = (B, N // tile_n)

    full = lambda shape: pl.BlockSpec(shape, lambda b, n: (0,) * len(shape))
    return pl.pallas_call(
        _fused_mlp_kernel,
        grid=grid,
        in_specs=[
            pl.BlockSpec((1, C_in, tile_n), lambda b, n: (b, 0, n)),
            full(w1.shape), full((w1.shape[0], 1)),
            full(w2.shape), full((w2.shape[0], 1)),
            full(w3.shape), full((w3.shape[0], 1)),
        ],
        out_specs=pl.BlockSpec((1, C_out, tile_n), lambda b, n: (b, 0, n)),
        out_shape=jax.ShapeDtypeStruct((B, C_out, N), jnp.float32),
        compiler_params=pltpu.CompilerParams(
            dimension_semantics=("parallel", "parallel")),
    )(x, w1, b1[:, None], w2, b2[:, None], w3, b3[:, None])


# speedup vs baseline: 2.7350x; 1.0843x over previous
import jax
import jax.numpy as jnp
from jax.experimental import pallas as pl
from jax.experimental.pallas import tpu as pltpu

_TILE_N = 2048


def _fused_mlp_kernel(x_ref, w1_ref, b1_ref, w2_ref, b2_ref, w3_ref, b3_ref,
                      o_ref):
    x = x_ref[0]
    dot = lambda w, h: jax.lax.dot_general(
        w, h, (((1,), (0,)), ((), ())), preferred_element_type=jnp.float32)
    h = jnp.maximum(dot(w1_ref[...], x) + b1_ref[...], 0.0)
    h = jnp.maximum(dot(w2_ref[...], h) + b2_ref[...], 0.0)
    o_ref[0] = jnp.maximum(dot(w3_ref[...], h) + b3_ref[...], 0.0)


def kernel(x, scale0_params, scale1_params, scale2_params, fusion_params):
    del scale0_params, scale1_params, scale2_params
    (w1, b1), (w2, b2), (w3, b3) = fusion_params
    B, C_in, N = x.shape
    C_out = w3.shape[0]
    tile_n = min(_TILE_N, N)
    grid = (B, N // tile_n)

    full = lambda shape: pl.BlockSpec(shape, lambda b, n: (0,) * len(shape))
    return pl.pallas_call(
        _fused_mlp_kernel,
        grid=grid,
        in_specs=[
            pl.BlockSpec((1, C_in, tile_n), lambda b, n: (b, 0, n)),
            full(w1.shape), full((w1.shape[0], 1)),
            full(w2.shape), full((w2.shape[0], 1)),
            full(w3.shape), full((w3.shape[0], 1)),
        ],
        out_specs=pl.BlockSpec((1, C_out, tile_n), lambda b, n: (b, 0, n)),
        out_shape=jax.ShapeDtypeStruct((B, C_out, N), jnp.float32),
        compiler_params=pltpu.CompilerParams(
            dimension_semantics=("parallel", "parallel")),
    )(x, w1, b1[:, None], w2, b2[:, None], w3, b3[:, None])
